# baseline (speedup 1.0000x reference)
import functools

import jax
import jax.numpy as jnp
from jax import lax
from jax.experimental import pallas as pl
from jax.experimental.pallas import tpu as pltpu
from jax.experimental.pallas import tpu_sc as plsc

_B = 16384
_D = 64
_L = 16


@functools.cache
def _build():
    info = plsc.get_sparse_core_info()
    nc, ns = info.num_cores, info.num_subcores
    nw = nc * ns
    bpw = _B // nw
    chunk = bpw // 2

    mesh = plsc.VectorSubcoreMesh(core_axis_name="c", subcore_axis_name="s")

    @functools.partial(
        pl.kernel,
        mesh=mesh,
        compiler_params=pltpu.CompilerParams(needs_layout_passes=False),
        out_type=jax.ShapeDtypeStruct((_B,), jnp.float32),
        scratch_types=[
            pltpu.VMEM((bpw,), jnp.int32),
            pltpu.VMEM((bpw,), jnp.int32),
            pltpu.VMEM((chunk // 2, 128), jnp.float32),
            pltpu.VMEM((chunk // 2, 128), jnp.float32),
            pltpu.VMEM((chunk // 8, 128), jnp.float32),
            pltpu.VMEM((chunk // 8, 128), jnp.float32),
            pltpu.VMEM((bpw,), jnp.float32),
            pltpu.SemaphoreType.DMA,
        ],
    )
    def sc_kernel(uids_hbm, aids_hbm, uemb_hbm, aemb_hbm, ub_hbm, ab_hbm,
                  out_hbm, uidx, aidx, urows, arows, ubv, abv, outv,
                  sem_rows):
        wid = lax.axis_index("s") * nc + lax.axis_index("c")
        base = wid * bpw
        pltpu.sync_copy(uids_hbm.at[pl.ds(base, bpw)], uidx)
        pltpu.sync_copy(aids_hbm.at[pl.ds(base, bpw)], aidx)

        lane = lax.iota(jnp.int32, _L)
        zeros = jnp.zeros((_L,), jnp.float32)

        for half in range(2):
            off = half * chunk

            def issue_body(g, carry, off=off):
                uvec = uidx[pl.ds(off + g * _L, _L)]
                avec = aidx[pl.ds(off + g * _L, _L)]
                for r in range(_L):
                    q = g * (_L // 2) + r // 2
                    rc = 64 * (r % 2)
                    b = g * 2 + r // 8
                    bc = 16 * (r % 8)
                    pltpu.async_copy(uemb_hbm.at[uvec[r]],
                                     urows.at[q, pl.ds(rc, _D)], sem_rows)
                    pltpu.async_copy(aemb_hbm.at[avec[r]],
                                     arows.at[q, pl.ds(rc, _D)], sem_rows)
                    pltpu.async_copy(ub_hbm.at[uvec[r]],
                                     ubv.at[b, pl.ds(bc, 1)], sem_rows)
                    pltpu.async_copy(ab_hbm.at[avec[r]],
                                     abv.at[b, pl.ds(bc, 1)], sem_rows)
                return carry

            lax.fori_loop(0, chunk // _L, issue_body, 0)

            def drain_body(i, carry):
                pltpu.make_async_copy(uemb_hbm.at[0],
                                      urows.at[0, pl.ds(0, _D)],
                                      sem_rows).wait()
                pltpu.make_async_copy(aemb_hbm.at[0],
                                      arows.at[0, pl.ds(0, _D)],
                                      sem_rows).wait()
                pltpu.make_async_copy(ub_hbm.at[0],
                                      ubv.at[0, pl.ds(0, 1)],
                                      sem_rows).wait()
                pltpu.make_async_copy(ab_hbm.at[0],
                                      abv.at[0, pl.ds(0, 1)],
                                      sem_rows).wait()
                return carry

            lax.fori_loop(0, chunk, drain_body, 0)

            def dot_body(g, carry, off=off):
                sl = pl.ds(off + g * _L, _L)
                acc = zeros
                for r in range(_L):
                    q = g * (_L // 2) + r // 2
                    rc = 64 * (r % 2)
                    b = g * 2 + r // 8
                    bc = 16 * (r % 8)
                    p = (urows[q, pl.ds(rc, _L)] *
                         arows[q, pl.ds(rc, _L)])
                    for j in range(1, _D // _L):
                        p = p + (urows[q, pl.ds(rc + j * _L, _L)] *
                                 arows[q, pl.ds(rc + j * _L, _L)])
                    s = jnp.sum(p) + ubv[b, pl.ds(bc, _L)][0] + \
                        abv[b, pl.ds(bc, _L)][0]
                    acc = jnp.where(lane == r, s, acc)
                outv[sl] = acc
                return carry

            lax.fori_loop(0, chunk // _L, dot_body, 0)

        pltpu.sync_copy(outv, out_hbm.at[pl.ds(base, bpw)])

    return sc_kernel


def kernel(userIds, animeIds, user_embeddings, anime_embeddings,
           user_biases, anime_biases):
    uids = userIds.astype(jnp.int32)
    aids = animeIds.astype(jnp.int32)
    return _build()(uids, aids, user_embeddings, anime_embeddings,
                    user_biases, anime_biases)


# speedup vs baseline: 1.0015x; 1.0015x over previous
import functools

import jax
import jax.numpy as jnp
from jax import lax
from jax.experimental import pallas as pl
from jax.experimental.pallas import tpu as pltpu
from jax.experimental.pallas import tpu_sc as plsc

_B = 16384
_D = 64
_L = 16


@functools.cache
def _build():
    info = plsc.get_sparse_core_info()
    nc, ns = info.num_cores, info.num_subcores
    nw = nc * ns
    bpw = _B // nw
    chunk = bpw // 2

    mesh = plsc.VectorSubcoreMesh(core_axis_name="c", subcore_axis_name="s")

    @functools.partial(
        pl.kernel,
        mesh=mesh,
        compiler_params=pltpu.CompilerParams(needs_layout_passes=False),
        out_type=jax.ShapeDtypeStruct((_B,), jnp.float32),
        scratch_types=[
            pltpu.VMEM((bpw,), jnp.int32),
            pltpu.VMEM((bpw,), jnp.int32),
            pltpu.VMEM((chunk // 2, 128), jnp.float32),
            pltpu.VMEM((chunk // 2, 128), jnp.float32),
            pltpu.VMEM((chunk // 8, 128), jnp.float32),
            pltpu.VMEM((chunk // 8, 128), jnp.float32),
            pltpu.VMEM((bpw,), jnp.float32),
            pltpu.SemaphoreType.DMA,
            pltpu.SemaphoreType.DMA,
            pltpu.SemaphoreType.DMA,
            pltpu.SemaphoreType.DMA,
            pltpu.SemaphoreType.DMA,
            pltpu.SemaphoreType.DMA,
            pltpu.SemaphoreType.DMA,
            pltpu.SemaphoreType.DMA,
        ],
    )
    def sc_kernel(uids_hbm, aids_hbm, uemb_hbm, aemb_hbm, ub_hbm, ab_hbm,
                  out_hbm, uidx, aidx, urows, arows, ubv, abv, outv,
                  *sems):
        wid = lax.axis_index("s") * nc + lax.axis_index("c")
        base = wid * bpw
        pltpu.sync_copy(uids_hbm.at[pl.ds(base, bpw)], uidx)
        pltpu.sync_copy(aids_hbm.at[pl.ds(base, bpw)], aidx)

        lane = lax.iota(jnp.int32, _L)
        zeros = jnp.zeros((_L,), jnp.float32)

        for half in range(2):
            off = half * chunk

            def issue_body(g, carry, off=off):
                uvec = uidx[pl.ds(off + g * _L, _L)]
                avec = aidx[pl.ds(off + g * _L, _L)]
                for r in range(_L):
                    q = g * (_L // 2) + r // 2
                    rc = 64 * (r % 2)
                    b = g * 2 + r // 8
                    bc = 16 * (r % 8)
                    sem = sems[r % 8]
                    pltpu.async_copy(uemb_hbm.at[uvec[r]],
                                     urows.at[q, pl.ds(rc, _D)], sem)
                    pltpu.async_copy(aemb_hbm.at[avec[r]],
                                     arows.at[q, pl.ds(rc, _D)], sem)
                    pltpu.async_copy(ub_hbm.at[uvec[r]],
                                     ubv.at[b, pl.ds(bc, 1)], sem)
                    pltpu.async_copy(ab_hbm.at[avec[r]],
                                     abv.at[b, pl.ds(bc, 1)], sem)
                return carry

            lax.fori_loop(0, chunk // _L, issue_body, 0)

            def drain_body(i, carry):
                for k in range(8):
                    sem = sems[k]
                    pltpu.make_async_copy(uemb_hbm.at[0],
                                          urows.at[0, pl.ds(0, _D)],
                                          sem).wait()
                    pltpu.make_async_copy(aemb_hbm.at[0],
                                          arows.at[0, pl.ds(0, _D)],
                                          sem).wait()
                    pltpu.make_async_copy(ub_hbm.at[0],
                                          ubv.at[0, pl.ds(0, 1)],
                                          sem).wait()
                    pltpu.make_async_copy(ab_hbm.at[0],
                                          abv.at[0, pl.ds(0, 1)],
                                          sem).wait()
                return carry

            lax.fori_loop(0, chunk // 8, drain_body, 0)

            def dot_body(g, carry, off=off):
                sl = pl.ds(off + g * _L, _L)
                acc = zeros
                for r in range(_L):
                    q = g * (_L // 2) + r // 2
                    rc = 64 * (r % 2)
                    b = g * 2 + r // 8
                    bc = 16 * (r % 8)
                    p = (urows[q, pl.ds(rc, _L)] *
                         arows[q, pl.ds(rc, _L)])
                    for j in range(1, _D // _L):
                        p = p + (urows[q, pl.ds(rc + j * _L, _L)] *
                                 arows[q, pl.ds(rc + j * _L, _L)])
                    s = jnp.sum(p) + ubv[b, pl.ds(bc, _L)][0] + \
                        abv[b, pl.ds(bc, _L)][0]
                    acc = jnp.where(lane == r, s, acc)
                outv[sl] = acc
                return carry

            lax.fori_loop(0, chunk // _L, dot_body, 0)

        pltpu.sync_copy(outv, out_hbm.at[pl.ds(base, bpw)])

    return sc_kernel


def kernel(userIds, animeIds, user_embeddings, anime_embeddings,
           user_biases, anime_biases):
    uids = userIds.astype(jnp.int32)
    aids = animeIds.astype(jnp.int32)
    return _build()(uids, aids, user_embeddings, anime_embeddings,
                    user_biases, anime_biases)
